# Initial kernel scaffold; baseline (speedup 1.0000x reference)
import jax
import jax.numpy as jnp
from jax.experimental import pallas as pl


def kernel(similarity, select):
    raise NotImplementedError("write your pallas kernel here")


# speedup vs baseline: 2.6621x; 2.6621x over previous
import functools

import jax
import jax.numpy as jnp
from jax.experimental import pallas as pl
from jax.experimental.pallas import tpu as pltpu

_B = 4096
_N = 8192
_TEMP = 0.1
_NEG_FILL = -1e30
_RB = 256


def _body(sim_ref, sel_ref, out_ref):
    i = pl.program_id(0)
    inv_t = 1.0 / _TEMP
    sim = sim_ref[...]
    sf = sel_ref[...].astype(jnp.float32)
    pos_sum = jnp.sum(sim * sf, axis=1)
    cnt = jnp.sum(sf, axis=1)
    pos_logit = (pos_sum / cnt) * inv_t
    neg = sim * inv_t + sf * (_NEG_FILL * inv_t)
    m = jnp.max(neg, axis=1)
    big = jnp.maximum(m, pos_logit)
    s = jnp.sum(jnp.exp(neg - big[:, None]), axis=1) + jnp.exp(pos_logit - big)
    loss = jnp.log(s) + big - pos_logit
    part = jnp.sum(loss) * (1.0 / _B)

    @pl.when(i == 0)
    def _init():
        out_ref[0, 0] = 0.0

    out_ref[0, 0] += part


@jax.jit
def kernel(similarity, select):
    out = pl.pallas_call(
        _body,
        grid=(_B // _RB,),
        in_specs=[
            pl.BlockSpec((_RB, _N), lambda i: (i, 0)),
            pl.BlockSpec((_RB, _N), lambda i: (i, 0)),
        ],
        out_specs=pl.BlockSpec(
            (1, 1), lambda i: (0, 0), memory_space=pltpu.SMEM
        ),
        out_shape=jax.ShapeDtypeStruct((1, 1), jnp.float32),
    )(similarity, select)
    return out[0, 0]
